# baseline (device time: 130309 ns/iter reference)
import jax
import jax.numpy as jnp
from jax import lax
from jax.experimental import pallas as pl
from jax.experimental.pallas import tpu as pltpu

_CHUNKS = [32, 32, 64, 128, 256] + [512] * 6 + [256, 128, 64, 32, 32]
_OFFS = [sum(_CHUNKS[:i]) for i in range(len(_CHUNKS))]


def kernel(x):
    m_shard, n = x.shape
    m_half = m_shard // 2
    assert sum(_CHUNKS) == m_half
    n_comm = len(_CHUNKS)
    lc = 512
    n_load = n_comm + m_half // lc
    n_slots = 4

    def body(x_ref, out_ref, mine, stage, load_sems, copy_sem, s1_sems,
             r1_sems, s2_sems, r2_sems):
        my_x = lax.axis_index("x")
        my_y = lax.axis_index("y")

        barrier_sem = pltpu.get_barrier_semaphore()
        pl.semaphore_signal(
            barrier_sem, inc=1,
            device_id=(my_x, 1 - my_y), device_id_type=pl.DeviceIdType.MESH,
        )
        pl.semaphore_signal(
            barrier_sem, inc=1,
            device_id=(1 - my_x, my_y), device_id_type=pl.DeviceIdType.MESH,
        )

        def load_spec(j):
            if j < n_comm:
                return my_x * m_half + _OFFS[j], _CHUNKS[j]
            return (1 - my_x) * m_half + (j - n_comm) * lc, lc

        def start_load(j):
            off, rows = load_spec(j)
            slot = (j % n_slots) * lc
            dma = pltpu.make_async_copy(
                x_ref.at[pl.ds(off, rows), :],
                stage.at[pl.ds(slot, rows), :],
                load_sems.at[j % n_slots],
            )
            dma.start()
            return dma

        loads = {j: start_load(j) for j in range(n_slots)}

        off1 = my_y * m_shard + my_x * m_half
        off2 = (1 - my_y) * m_shard + my_x * m_half

        pl.semaphore_wait(barrier_sem, 2)

        p1 = []
        for j in range(n_load):
            loads[j].wait()
            off, rows = load_spec(j)
            slot = (j % n_slots) * lc
            mine[pl.ds(off, rows), :] = stage[
                pl.ds(slot, rows), :].astype(mine.dtype)
            if j + n_slots < n_load:
                loads[j + n_slots] = start_load(j + n_slots)
            if j < n_comm:
                rdma = pltpu.make_async_remote_copy(
                    src_ref=mine.at[pl.ds(off, rows), :],
                    dst_ref=out_ref.at[pl.ds(off1 + _OFFS[j], rows), :],
                    send_sem=s1_sems.at[j],
                    recv_sem=r1_sems.at[j],
                    device_id=(my_x, 1 - my_y),
                    device_id_type=pl.DeviceIdType.MESH,
                )
                rdma.start()
                p1.append(rdma)

        own = pltpu.make_async_copy(
            mine,
            out_ref.at[pl.ds(my_y * m_shard, m_shard), :],
            copy_sem,
        )
        own.start()

        p2 = []
        for j in range(n_comm):
            p1[j].wait_recv()
            rdma = pltpu.make_async_remote_copy(
                src_ref=out_ref.at[pl.ds(off2 + _OFFS[j], _CHUNKS[j]), :],
                dst_ref=out_ref.at[pl.ds(off2 + _OFFS[j], _CHUNKS[j]), :],
                send_sem=s2_sems.at[j],
                recv_sem=r2_sems.at[j],
                device_id=(1 - my_x, my_y),
                device_id_type=pl.DeviceIdType.MESH,
            )
            rdma.start()
            p2.append(rdma)

        for j in range(n_comm):
            p2[j].wait_recv()
        for j in range(n_comm):
            p1[j].wait_send()
            p2[j].wait_send()
        own.wait()

    return pl.pallas_call(
        body,
        out_shape=jax.ShapeDtypeStruct((2 * m_shard, n), jnp.bfloat16),
        in_specs=[pl.BlockSpec(memory_space=pltpu.MemorySpace.HBM)],
        out_specs=pl.BlockSpec(memory_space=pltpu.MemorySpace.HBM),
        scratch_shapes=[
            pltpu.VMEM((m_shard, n), jnp.bfloat16),
            pltpu.VMEM((n_slots * lc, n), jnp.float32),
            pltpu.SemaphoreType.DMA((n_slots,)),
            pltpu.SemaphoreType.DMA,
            pltpu.SemaphoreType.DMA((n_comm,)),
            pltpu.SemaphoreType.DMA((n_comm,)),
            pltpu.SemaphoreType.DMA((n_comm,)),
            pltpu.SemaphoreType.DMA((n_comm,)),
        ],
        compiler_params=pltpu.CompilerParams(
            collective_id=0,
            vmem_limit_bytes=60 * 1024 * 1024,
        ),
    )(x)


# device time: 130075 ns/iter; 1.0018x vs baseline; 1.0018x over previous
import jax
import jax.numpy as jnp
from jax import lax
from jax.experimental import pallas as pl
from jax.experimental.pallas import tpu as pltpu

_CHUNKS = [32, 32, 64, 128, 256] + [512] * 6 + [256, 128, 64, 32, 32]
_OFFS = [sum(_CHUNKS[:i]) for i in range(len(_CHUNKS))]


def kernel(x):
    m_shard, n = x.shape
    m_half = m_shard // 2
    assert sum(_CHUNKS) == m_half
    n_comm = len(_CHUNKS)
    lc = 512
    n_load = n_comm + m_half // lc
    n_slots = 4

    def body(x_ref, out_ref, mine, stage, recv1, load_sems, copy_sem,
             c1_sems, s1_sems, r1_sems, s2_sems, r2_sems):
        my_x = lax.axis_index("x")
        my_y = lax.axis_index("y")

        barrier_sem = pltpu.get_barrier_semaphore()
        pl.semaphore_signal(
            barrier_sem, inc=1,
            device_id=(my_x, 1 - my_y), device_id_type=pl.DeviceIdType.MESH,
        )
        pl.semaphore_signal(
            barrier_sem, inc=1,
            device_id=(1 - my_x, my_y), device_id_type=pl.DeviceIdType.MESH,
        )

        def load_spec(j):
            if j < n_comm:
                return my_x * m_half + _OFFS[j], _CHUNKS[j]
            return (1 - my_x) * m_half + (j - n_comm) * lc, lc

        def start_load(j):
            off, rows = load_spec(j)
            slot = (j % n_slots) * lc
            dma = pltpu.make_async_copy(
                x_ref.at[pl.ds(off, rows), :],
                stage.at[pl.ds(slot, rows), :],
                load_sems.at[j % n_slots],
            )
            dma.start()
            return dma

        loads = {j: start_load(j) for j in range(n_slots)}

        off1 = my_y * m_shard + my_x * m_half
        off2 = (1 - my_y) * m_shard + my_x * m_half

        pl.semaphore_wait(barrier_sem, 2)

        p1 = []
        for j in range(n_load):
            loads[j].wait()
            off, rows = load_spec(j)
            slot = (j % n_slots) * lc
            mine[pl.ds(off, rows), :] = stage[
                pl.ds(slot, rows), :].astype(mine.dtype)
            if j + n_slots < n_load:
                loads[j + n_slots] = start_load(j + n_slots)
            if j < n_comm:
                rdma = pltpu.make_async_remote_copy(
                    src_ref=mine.at[pl.ds(off, rows), :],
                    dst_ref=recv1.at[pl.ds(_OFFS[j], rows), :],
                    send_sem=s1_sems.at[j],
                    recv_sem=r1_sems.at[j],
                    device_id=(my_x, 1 - my_y),
                    device_id_type=pl.DeviceIdType.MESH,
                )
                rdma.start()
                p1.append(rdma)

        own = pltpu.make_async_copy(
            mine,
            out_ref.at[pl.ds(my_y * m_shard, m_shard), :],
            copy_sem,
        )
        own.start()

        p2 = []
        copies = []
        for j in range(n_comm):
            p1[j].wait_recv()
            rdma = pltpu.make_async_remote_copy(
                src_ref=recv1.at[pl.ds(_OFFS[j], _CHUNKS[j]), :],
                dst_ref=out_ref.at[pl.ds(off2 + _OFFS[j], _CHUNKS[j]), :],
                send_sem=s2_sems.at[j],
                recv_sem=r2_sems.at[j],
                device_id=(1 - my_x, my_y),
                device_id_type=pl.DeviceIdType.MESH,
            )
            rdma.start()
            p2.append(rdma)
            c = pltpu.make_async_copy(
                recv1.at[pl.ds(_OFFS[j], _CHUNKS[j]), :],
                out_ref.at[pl.ds(off2 + _OFFS[j], _CHUNKS[j]), :],
                c1_sems.at[j],
            )
            c.start()
            copies.append(c)

        for j in range(n_comm):
            p2[j].wait_recv()
        for j in range(n_comm):
            p1[j].wait_send()
            p2[j].wait_send()
            copies[j].wait()
        own.wait()

    return pl.pallas_call(
        body,
        out_shape=jax.ShapeDtypeStruct((2 * m_shard, n), jnp.bfloat16),
        in_specs=[pl.BlockSpec(memory_space=pltpu.MemorySpace.HBM)],
        out_specs=pl.BlockSpec(memory_space=pltpu.MemorySpace.HBM),
        scratch_shapes=[
            pltpu.VMEM((m_shard, n), jnp.bfloat16),
            pltpu.VMEM((n_slots * lc, n), jnp.float32),
            pltpu.VMEM((m_half, n), jnp.bfloat16),
            pltpu.SemaphoreType.DMA((n_slots,)),
            pltpu.SemaphoreType.DMA,
            pltpu.SemaphoreType.DMA((n_comm,)),
            pltpu.SemaphoreType.DMA((n_comm,)),
            pltpu.SemaphoreType.DMA((n_comm,)),
            pltpu.SemaphoreType.DMA((n_comm,)),
            pltpu.SemaphoreType.DMA((n_comm,)),
        ],
        compiler_params=pltpu.CompilerParams(
            collective_id=0,
            vmem_limit_bytes=60 * 1024 * 1024,
        ),
    )(x)
